# baseline (speedup 1.0000x reference)
import functools
import math

import numpy as np
import jax
import jax.numpy as jnp
from jax import lax
from jax.experimental import pallas as pl
from jax.experimental.pallas import tpu as pltpu
from jax.experimental.pallas import tpu_sc as plsc

DIM = 128
BITS = 4
EPS = 1e-08
L = 16
VPR = DIM // L
NC, NS = 2, 16
NW = NC * NS


def _lm_centroids(bits, iters=100):
    n = 2 ** bits
    xs = np.linspace(-8.0, 8.0, 200001)
    pdf = np.exp(-0.5 * xs ** 2)
    cdf = np.cumsum(pdf)
    cdf = cdf / cdf[-1]
    c = np.interp((np.arange(n) + 0.5) / n, cdf, xs)
    for _ in range(iters):
        b = 0.5 * (c[:-1] + c[1:])
        idx = np.searchsorted(b, xs)
        num = np.bincount(idx, weights=pdf * xs, minlength=n)
        den = np.bincount(idx, weights=pdf, minlength=n)
        c = np.where(den > 1e-12, num / np.maximum(den, 1e-12), c)
    return c.astype(np.float32)


_CEN = _lm_centroids(BITS)
_BND = (0.5 * (_CEN[:-1] + _CEN[1:])).astype(np.float32)
_BND16 = np.concatenate([_BND, [np.float32(np.inf)]])
_MAXC = float(_CEN.max())
_SGN = (np.random.RandomState(42).randint(0, 2, (1, DIM)) * 2 - 1).astype(
    np.float32)[0]
_S = 1.0 / math.sqrt(float(DIM))

_CONSTS = np.concatenate(
    [_CEN, _BND16, (_SGN * np.float32(_S)).astype(np.float32)]
).astype(np.float32)

_GDN = lax.GatherDimensionNumbers(
    offset_dims=(), collapsed_slice_dims=(0,), start_index_map=(0,))


def _dg(vec, idx):
    return lax.gather(vec, idx[:, None], _GDN, (1,),
                      mode=lax.GatherScatterMode.PROMISE_IN_BOUNDS)


def _lane_sum(v):
    lane = lax.iota(jnp.int32, L)
    for h in (1, 2, 4, 8):
        v = v + _dg(v, lane ^ h)
    return v


def _lane_max(v):
    lane = lax.iota(jnp.int32, L)
    for h in (1, 2, 4, 8):
        v = jnp.maximum(v, _dg(v, lane ^ h))
    return v


def _fwht_regs(v):
    lane = lax.iota(jnp.int32, L)
    for h in (1, 2, 4, 8):
        perm = lane ^ h
        pm = jnp.where((lane & h) == 0, 1.0, -1.0).astype(jnp.float32)
        v = [_dg(x, perm) + pm * x for x in v]
    for hb in (1, 2, 4):
        nv = list(v)
        for j in range(VPR):
            if j & hb == 0:
                nv[j] = v[j] + v[j ^ hb]
            else:
                nv[j] = v[j ^ hb] - v[j]
        v = nv
    return v


def _quant_pass(u, bs, cvec, keep):
    recon = [] if keep else None
    num_acc = None
    den_acc = None
    for j in range(VPR):
        idx = jnp.zeros((L,), jnp.int32)
        for stp in (8, 4, 2, 1):
            bv = _dg(bs, idx + (stp - 1))
            idx = idx + jnp.where(bv < u[j], stp, 0).astype(jnp.int32)
        r = _dg(cvec, idx)
        if keep:
            recon.append(r)
        nj = u[j] * r
        dj = r * r
        num_acc = nj if num_acc is None else num_acc + nj
        den_acc = dj if den_acc is None else den_acc + dj
    return recon, _lane_sum(num_acc), _lane_sum(den_acc)


_H = np.ones((1, 1), np.float32)
for _ in range(7):
    _H = np.block([[_H, _H], [_H, -_H]])
_M = (np.diag(_SGN) @ _H * np.float32(_S)).astype(np.float32)
_SQD = float(math.sqrt(float(DIM)))


def _tc_recon(un, d):
    r = jnp.full(un.shape, float(_CEN[0]), jnp.float32)
    for k in range(15):
        r = r + jnp.where(un > float(_BND[k]) * d,
                          float(_CEN[k + 1] - _CEN[k]), 0.0)
    return r


def _tc_body(x_ref, m_ref, o_ref):
    xt = x_ref[...]
    m = m_ref[...]
    u = jnp.dot(xt, m, preferred_element_type=jnp.float32,
                precision=lax.Precision.HIGHEST)
    au = jnp.abs(u)
    maxu = jnp.max(au, axis=-1, keepdims=True)
    sumau = jnp.sum(au, axis=-1, keepdims=True)
    t = EPS * (maxu * _SQD + EPS)
    grms = maxu * (1.0 / _MAXC)
    d1 = grms + t
    r1 = _tc_recon(u, d1)
    num1 = jnp.sum(u * r1, axis=-1, keepdims=True)
    den1 = jnp.sum(r1 * r1, axis=-1, keepdims=True)
    d2 = num1 / (den1 + EPS) + t
    spiky = maxu > 5.0 * (sumau * (1.0 / DIM) + t)
    df = jnp.where(spiky, d1, d2)
    rf = _tc_recon(u, df)
    num2 = jnp.sum(u * rf, axis=-1, keepdims=True)
    den2 = jnp.sum(rf * rf, axis=-1, keepdims=True)
    gain = jnp.where(spiky, grms, num2 / (den2 + EPS))
    w = jnp.dot(rf, m.T, preferred_element_type=jnp.float32,
                precision=lax.Precision.HIGHEST)
    o_ref[...] = w * gain


def _make_tc_call(rows, tile):
    grid = rows // tile
    return pl.pallas_call(
        _tc_body,
        grid=(grid,),
        in_specs=[
            pl.BlockSpec((tile, DIM), lambda i: (i, 0)),
            pl.BlockSpec((DIM, DIM), lambda i: (0, 0)),
        ],
        out_specs=pl.BlockSpec((tile, DIM), lambda i: (i, 0)),
        out_shape=jax.ShapeDtypeStruct((rows, DIM), jnp.float32),
    )


def _make_sc_call(rows, ch, interpret=False):
    rpw = rows // NW
    nch = rpw // ch
    mesh = plsc.VectorSubcoreMesh(core_axis_name="c", subcore_axis_name="s",
                                  num_cores=NC, num_subcores=NS)

    @functools.partial(
        pl.kernel,
        out_type=jax.ShapeDtypeStruct((rows * DIM,), jnp.float32),
        mesh=mesh,
        scratch_types=[
            pltpu.VMEM((_CONSTS.size,), jnp.float32),
            pltpu.VMEM((ch * DIM,), jnp.float32),
            pltpu.VMEM((ch * DIM,), jnp.float32),
        ],
        interpret=interpret,
    )
    def sc_fn(x_hbm, c_hbm, o_hbm, cbuf, inb, outb):
        wid = lax.axis_index("s") * NC + lax.axis_index("c")
        base = wid * rpw * DIM

        pltpu.sync_copy(c_hbm, cbuf)
        cvec = cbuf[pl.ds(0, L)]
        bvec = cbuf[pl.ds(L, L)]
        sgn_s = [cbuf[pl.ds(2 * L + L * j, L)] for j in range(VPR)]

        def row_body(i, carry):
            ro = i * DIM
            v = [inb[pl.ds(ro + L * j, L)] for j in range(VPR)]
            u = _fwht_regs([v[j] * sgn_s[j] for j in range(VPR)])
            au = [jnp.abs(t) for t in u]
            mx = au[0]
            sa = au[0]
            for j in range(1, VPR):
                mx = jnp.maximum(mx, au[j])
                sa = sa + au[j]
            maxu = _lane_max(mx)
            sumau = _lane_sum(sa)

            t = EPS * (maxu * math.sqrt(float(DIM)) + EPS)
            grms = maxu * (1.0 / _MAXC)
            d1 = grms + t
            _, num1, den1 = _quant_pass(u, bvec * d1, cvec, keep=False)
            g1p = num1 / (den1 + EPS)
            d2 = g1p + t
            spiky = maxu > 5.0 * (sumau * (1.0 / DIM) + t)
            df = jnp.where(spiky, d1, d2)
            rec, num2, den2 = _quant_pass(u, bvec * df, cvec, keep=True)
            gain = jnp.where(spiky, grms, num2 / (den2 + EPS))

            w = _fwht_regs(rec)
            for j in range(VPR):
                outb[pl.ds(ro + L * j, L)] = w[j] * sgn_s[j] * gain
            return carry

        def chunk_body(ci, carry):
            off = base + ci * (ch * DIM)
            pltpu.sync_copy(x_hbm.at[pl.ds(off, ch * DIM)], inb)
            lax.fori_loop(0, ch, row_body, 0)
            pltpu.sync_copy(outb, o_hbm.at[pl.ds(off, ch * DIM)])
            return carry

        lax.fori_loop(0, nch, chunk_body, 0)

    return sc_fn


def kernel(x):
    shape = x.shape
    rows = x.size // DIM
    xf = x.astype(jnp.float32).reshape(rows, DIM)
    rows_sc = (rows * 5 // 8) // (NW * 64) * (NW * 64)
    if rows_sc == 0:
        rows_sc = rows
    rpw = rows_sc // NW
    ch = 64 if rpw % 64 == 0 else rpw
    out_sc = _make_sc_call(rows_sc, ch)(
        xf[:rows_sc].reshape(-1), jnp.asarray(_CONSTS))
    parts = [out_sc.reshape(rows_sc, DIM)]
    rows_tc = rows - rows_sc
    if rows_tc:
        tile = 512 if rows_tc % 512 == 0 else rows_tc
        parts.append(_make_tc_call(rows_tc, tile)(
            xf[rows_sc:], jnp.asarray(_M)))
    out = jnp.concatenate(parts, axis=0) if len(parts) > 1 else parts[0]
    return out.reshape(shape)


# speedup vs baseline: 1.1884x; 1.1884x over previous
import functools
import math

import numpy as np
import jax
import jax.numpy as jnp
from jax import lax
from jax.experimental import pallas as pl
from jax.experimental.pallas import tpu as pltpu
from jax.experimental.pallas import tpu_sc as plsc

DIM = 128
BITS = 4
EPS = 1e-08
L = 16
VPR = DIM // L
NC, NS = 2, 16
NW = NC * NS


def _lm_centroids(bits, iters=100):
    n = 2 ** bits
    xs = np.linspace(-8.0, 8.0, 200001)
    pdf = np.exp(-0.5 * xs ** 2)
    cdf = np.cumsum(pdf)
    cdf = cdf / cdf[-1]
    c = np.interp((np.arange(n) + 0.5) / n, cdf, xs)
    for _ in range(iters):
        b = 0.5 * (c[:-1] + c[1:])
        idx = np.searchsorted(b, xs)
        num = np.bincount(idx, weights=pdf * xs, minlength=n)
        den = np.bincount(idx, weights=pdf, minlength=n)
        c = np.where(den > 1e-12, num / np.maximum(den, 1e-12), c)
    return c.astype(np.float32)


_CEN = _lm_centroids(BITS)
_BND = (0.5 * (_CEN[:-1] + _CEN[1:])).astype(np.float32)
_BND16 = np.concatenate([_BND, [np.float32(np.inf)]])
_MAXC = float(_CEN.max())
_SGN = (np.random.RandomState(42).randint(0, 2, (1, DIM)) * 2 - 1).astype(
    np.float32)[0]
_S = 1.0 / math.sqrt(float(DIM))

_CONSTS = np.concatenate(
    [_CEN, _BND16, (_SGN * np.float32(_S)).astype(np.float32)]
).astype(np.float32)

_GDN = lax.GatherDimensionNumbers(
    offset_dims=(), collapsed_slice_dims=(0,), start_index_map=(0,))


def _dg(vec, idx):
    return lax.gather(vec, idx[:, None], _GDN, (1,),
                      mode=lax.GatherScatterMode.PROMISE_IN_BOUNDS)


def _lane_sum(v):
    lane = lax.iota(jnp.int32, L)
    for h in (1, 2, 4, 8):
        v = v + _dg(v, lane ^ h)
    return v


def _lane_max(v):
    lane = lax.iota(jnp.int32, L)
    for h in (1, 2, 4, 8):
        v = jnp.maximum(v, _dg(v, lane ^ h))
    return v


def _fwht_regs(v):
    lane = lax.iota(jnp.int32, L)
    for h in (1, 2, 4, 8):
        perm = lane ^ h
        pm = jnp.where((lane & h) == 0, 1.0, -1.0).astype(jnp.float32)
        v = [_dg(x, perm) + pm * x for x in v]
    for hb in (1, 2, 4):
        nv = list(v)
        for j in range(VPR):
            if j & hb == 0:
                nv[j] = v[j] + v[j ^ hb]
            else:
                nv[j] = v[j ^ hb] - v[j]
        v = nv
    return v


def _quant_pass(u, bs, cvec, keep):
    recon = [] if keep else None
    num_acc = None
    den_acc = None
    for j in range(VPR):
        idx = jnp.zeros((L,), jnp.int32)
        for stp in (8, 4, 2, 1):
            bv = _dg(bs, idx + (stp - 1))
            idx = idx + jnp.where(bv < u[j], stp, 0).astype(jnp.int32)
        r = _dg(cvec, idx)
        if keep:
            recon.append(r)
        nj = u[j] * r
        dj = r * r
        num_acc = nj if num_acc is None else num_acc + nj
        den_acc = dj if den_acc is None else den_acc + dj
    return recon, _lane_sum(num_acc), _lane_sum(den_acc)


_H = np.ones((1, 1), np.float32)
for _ in range(7):
    _H = np.block([[_H, _H], [_H, -_H]])
_M = (np.diag(_SGN) @ _H * np.float32(_S)).astype(np.float32)
_SQD = float(math.sqrt(float(DIM)))


def _tc_recon(un, d):
    r = jnp.full(un.shape, float(_CEN[0]), jnp.float32)
    for k in range(15):
        r = r + jnp.where(un > float(_BND[k]) * d,
                          float(_CEN[k + 1] - _CEN[k]), 0.0)
    return r


def _tc_body(x_ref, m_ref, o_ref):
    xt = x_ref[...]
    m = m_ref[...]
    u = jnp.dot(xt, m, preferred_element_type=jnp.float32,
                precision=lax.Precision.HIGHEST)
    au = jnp.abs(u)
    maxu = jnp.max(au, axis=-1, keepdims=True)
    sumau = jnp.sum(au, axis=-1, keepdims=True)
    t = EPS * (maxu * _SQD + EPS)
    grms = maxu * (1.0 / _MAXC)
    d1 = grms + t
    r1 = _tc_recon(u, d1)
    num1 = jnp.sum(u * r1, axis=-1, keepdims=True)
    den1 = jnp.sum(r1 * r1, axis=-1, keepdims=True)
    d2 = num1 / (den1 + EPS) + t
    spiky = maxu > 5.0 * (sumau * (1.0 / DIM) + t)
    df = jnp.where(spiky, d1, d2)
    rf = _tc_recon(u, df)
    num2 = jnp.sum(u * rf, axis=-1, keepdims=True)
    den2 = jnp.sum(rf * rf, axis=-1, keepdims=True)
    gain = jnp.where(spiky, grms, num2 / (den2 + EPS))
    w = jnp.dot(rf, m.T, preferred_element_type=jnp.float32,
                precision=lax.Precision.HIGHEST)
    o_ref[...] = w * gain


def _make_tc_call(rows, tile):
    grid = rows // tile
    return pl.pallas_call(
        _tc_body,
        grid=(grid,),
        in_specs=[
            pl.BlockSpec((tile, DIM), lambda i: (i, 0)),
            pl.BlockSpec((DIM, DIM), lambda i: (0, 0)),
        ],
        out_specs=pl.BlockSpec((tile, DIM), lambda i: (i, 0)),
        out_shape=jax.ShapeDtypeStruct((rows, DIM), jnp.float32),
    )


def _make_sc_call(rows, ch, interpret=False):
    rpw = rows // NW
    nch = rpw // ch
    mesh = plsc.VectorSubcoreMesh(core_axis_name="c", subcore_axis_name="s",
                                  num_cores=NC, num_subcores=NS)

    @functools.partial(
        pl.kernel,
        out_type=jax.ShapeDtypeStruct((rows * DIM,), jnp.float32),
        mesh=mesh,
        scratch_types=[
            pltpu.VMEM((_CONSTS.size,), jnp.float32),
            pltpu.VMEM((ch * DIM,), jnp.float32),
            pltpu.VMEM((ch * DIM,), jnp.float32),
        ],
        interpret=interpret,
    )
    def sc_fn(x_hbm, c_hbm, o_hbm, cbuf, inb, outb):
        wid = lax.axis_index("s") * NC + lax.axis_index("c")
        base = wid * rpw * DIM

        pltpu.sync_copy(c_hbm, cbuf)
        cvec = cbuf[pl.ds(0, L)]
        bvec = cbuf[pl.ds(L, L)]
        sgn_s = [cbuf[pl.ds(2 * L + L * j, L)] for j in range(VPR)]

        def row_body(i, carry):
            ro = i * DIM
            v = [inb[pl.ds(ro + L * j, L)] for j in range(VPR)]
            u = _fwht_regs([v[j] * sgn_s[j] for j in range(VPR)])
            au = [jnp.abs(t) for t in u]
            mx = au[0]
            sa = au[0]
            for j in range(1, VPR):
                mx = jnp.maximum(mx, au[j])
                sa = sa + au[j]
            maxu = _lane_max(mx)
            sumau = _lane_sum(sa)

            t = EPS * (maxu * math.sqrt(float(DIM)) + EPS)
            grms = maxu * (1.0 / _MAXC)
            d1 = grms + t
            _, num1, den1 = _quant_pass(u, bvec * d1, cvec, keep=False)
            g1p = num1 / (den1 + EPS)
            d2 = g1p + t
            spiky = maxu > 5.0 * (sumau * (1.0 / DIM) + t)
            df = jnp.where(spiky, d1, d2)
            rec, num2, den2 = _quant_pass(u, bvec * df, cvec, keep=True)
            gain = jnp.where(spiky, grms, num2 / (den2 + EPS))

            w = _fwht_regs(rec)
            for j in range(VPR):
                outb[pl.ds(ro + L * j, L)] = w[j] * sgn_s[j] * gain
            return carry

        def chunk_body(ci, carry):
            off = base + ci * (ch * DIM)
            pltpu.sync_copy(x_hbm.at[pl.ds(off, ch * DIM)], inb)
            lax.fori_loop(0, ch, row_body, 0)
            pltpu.sync_copy(outb, o_hbm.at[pl.ds(off, ch * DIM)])
            return carry

        lax.fori_loop(0, nch, chunk_body, 0)

    return sc_fn


def kernel(x):
    shape = x.shape
    rows = x.size // DIM
    xf = x.astype(jnp.float32).reshape(rows, DIM)
    rows_sc = (rows // 2) // (NW * 64) * (NW * 64)
    if rows_sc == 0:
        rows_sc = rows
    rpw = rows_sc // NW
    ch = 64 if rpw % 64 == 0 else rpw
    rows_tc = rows - rows_sc
    out_tc = None
    if rows_tc:
        tile = 512 if rows_tc % 512 == 0 else rows_tc
        out_tc = _make_tc_call(rows_tc, tile)(xf[rows_sc:], jnp.asarray(_M))
    out_sc = _make_sc_call(rows_sc, ch)(
        xf[:rows_sc].reshape(-1), jnp.asarray(_CONSTS))
    parts = [out_sc.reshape(rows_sc, DIM)]
    if out_tc is not None:
        parts.append(out_tc)
    out = jnp.concatenate(parts, axis=0) if len(parts) > 1 else parts[0]
    return out.reshape(shape)


# speedup vs baseline: 1.1885x; 1.0000x over previous
import functools
import math

import numpy as np
import jax
import jax.numpy as jnp
from jax import lax
from jax.experimental import pallas as pl
from jax.experimental.pallas import tpu as pltpu
from jax.experimental.pallas import tpu_sc as plsc

DIM = 128
BITS = 4
EPS = 1e-08
L = 16
VPR = DIM // L
NC, NS = 2, 16
NW = NC * NS


def _lm_centroids(bits, iters=100):
    n = 2 ** bits
    xs = np.linspace(-8.0, 8.0, 200001)
    pdf = np.exp(-0.5 * xs ** 2)
    cdf = np.cumsum(pdf)
    cdf = cdf / cdf[-1]
    c = np.interp((np.arange(n) + 0.5) / n, cdf, xs)
    for _ in range(iters):
        b = 0.5 * (c[:-1] + c[1:])
        idx = np.searchsorted(b, xs)
        num = np.bincount(idx, weights=pdf * xs, minlength=n)
        den = np.bincount(idx, weights=pdf, minlength=n)
        c = np.where(den > 1e-12, num / np.maximum(den, 1e-12), c)
    return c.astype(np.float32)


_CEN = _lm_centroids(BITS)
_BND = (0.5 * (_CEN[:-1] + _CEN[1:])).astype(np.float32)
_BND16 = np.concatenate([_BND, [np.float32(np.inf)]])
_MAXC = float(_CEN.max())
_SGN = (np.random.RandomState(42).randint(0, 2, (1, DIM)) * 2 - 1).astype(
    np.float32)[0]
_S = 1.0 / math.sqrt(float(DIM))

_CONSTS = np.concatenate(
    [_CEN, _BND16, (_SGN * np.float32(_S)).astype(np.float32)]
).astype(np.float32)

_GDN = lax.GatherDimensionNumbers(
    offset_dims=(), collapsed_slice_dims=(0,), start_index_map=(0,))


def _dg(vec, idx):
    return lax.gather(vec, idx[:, None], _GDN, (1,),
                      mode=lax.GatherScatterMode.PROMISE_IN_BOUNDS)


def _lane_sum(v):
    lane = lax.iota(jnp.int32, L)
    for h in (1, 2, 4, 8):
        v = v + _dg(v, lane ^ h)
    return v


def _lane_max(v):
    lane = lax.iota(jnp.int32, L)
    for h in (1, 2, 4, 8):
        v = jnp.maximum(v, _dg(v, lane ^ h))
    return v


def _fwht_regs(v):
    lane = lax.iota(jnp.int32, L)
    for h in (1, 2, 4, 8):
        perm = lane ^ h
        pm = jnp.where((lane & h) == 0, 1.0, -1.0).astype(jnp.float32)
        v = [_dg(x, perm) + pm * x for x in v]
    for hb in (1, 2, 4):
        nv = list(v)
        for j in range(VPR):
            if j & hb == 0:
                nv[j] = v[j] + v[j ^ hb]
            else:
                nv[j] = v[j ^ hb] - v[j]
        v = nv
    return v


def _quant_pass(u, bs, cvec, keep):
    recon = [] if keep else None
    num_acc = None
    den_acc = None
    for j in range(VPR):
        idx = jnp.zeros((L,), jnp.int32)
        for stp in (8, 4, 2, 1):
            bv = _dg(bs, idx + (stp - 1))
            idx = idx + jnp.where(bv < u[j], stp, 0).astype(jnp.int32)
        r = _dg(cvec, idx)
        if keep:
            recon.append(r)
        nj = u[j] * r
        dj = r * r
        num_acc = nj if num_acc is None else num_acc + nj
        den_acc = dj if den_acc is None else den_acc + dj
    return recon, _lane_sum(num_acc), _lane_sum(den_acc)


_H = np.ones((1, 1), np.float32)
for _ in range(7):
    _H = np.block([[_H, _H], [_H, -_H]])
_M = (np.diag(_SGN) @ _H * np.float32(_S)).astype(np.float32)
_SQD = float(math.sqrt(float(DIM)))


def _tc_recon(un, d):
    r = jnp.full(un.shape, float(_CEN[0]), jnp.float32)
    for k in range(15):
        r = r + jnp.where(un > float(_BND[k]) * d,
                          float(_CEN[k + 1] - _CEN[k]), 0.0)
    return r


def _tc_body(x_ref, m_ref, o_ref):
    xt = x_ref[...]
    m = m_ref[...]
    u = jnp.dot(xt, m, preferred_element_type=jnp.float32,
                precision=lax.Precision.HIGHEST)
    au = jnp.abs(u)
    maxu = jnp.max(au, axis=-1, keepdims=True)
    sumau = jnp.sum(au, axis=-1, keepdims=True)
    t = EPS * (maxu * _SQD + EPS)
    grms = maxu * (1.0 / _MAXC)
    d1 = grms + t
    r1 = _tc_recon(u, d1)
    num1 = jnp.sum(u * r1, axis=-1, keepdims=True)
    den1 = jnp.sum(r1 * r1, axis=-1, keepdims=True)
    d2 = num1 / (den1 + EPS) + t
    spiky = maxu > 5.0 * (sumau * (1.0 / DIM) + t)
    df = jnp.where(spiky, d1, d2)
    rf = _tc_recon(u, df)
    num2 = jnp.sum(u * rf, axis=-1, keepdims=True)
    den2 = jnp.sum(rf * rf, axis=-1, keepdims=True)
    gain = jnp.where(spiky, grms, num2 / (den2 + EPS))
    w = jnp.dot(rf, m.T, preferred_element_type=jnp.float32,
                precision=lax.Precision.HIGHEST)
    o_ref[...] = w * gain


def _make_tc_call(rows, tile):
    grid = rows // tile
    return pl.pallas_call(
        _tc_body,
        grid=(grid,),
        in_specs=[
            pl.BlockSpec((tile, DIM), lambda i: (i, 0)),
            pl.BlockSpec((DIM, DIM), lambda i: (0, 0)),
        ],
        out_specs=pl.BlockSpec((tile, DIM), lambda i: (i, 0)),
        out_shape=jax.ShapeDtypeStruct((rows, DIM), jnp.float32),
    )


def _make_sc_call(rows, ch, interpret=False):
    rpw = rows // NW
    nch = rpw // ch
    mesh = plsc.VectorSubcoreMesh(core_axis_name="c", subcore_axis_name="s",
                                  num_cores=NC, num_subcores=NS)

    @functools.partial(
        pl.kernel,
        out_type=jax.ShapeDtypeStruct((rows * DIM,), jnp.float32),
        mesh=mesh,
        scratch_types=[
            pltpu.VMEM((_CONSTS.size,), jnp.float32),
            pltpu.VMEM((ch * DIM,), jnp.float32),
            pltpu.VMEM((ch * DIM,), jnp.float32),
        ],
        interpret=interpret,
    )
    def sc_fn(x_hbm, c_hbm, o_hbm, cbuf, inb, outb):
        wid = lax.axis_index("s") * NC + lax.axis_index("c")
        base = wid * rpw * DIM

        pltpu.sync_copy(c_hbm, cbuf)
        cvec = cbuf[pl.ds(0, L)]
        bvec = cbuf[pl.ds(L, L)]
        sgn_s = [cbuf[pl.ds(2 * L + L * j, L)] for j in range(VPR)]

        def row_body(i, carry):
            ro = i * DIM
            v = [inb[pl.ds(ro + L * j, L)] for j in range(VPR)]
            u = _fwht_regs([v[j] * sgn_s[j] for j in range(VPR)])
            au = [jnp.abs(t) for t in u]
            mx = au[0]
            sa = au[0]
            for j in range(1, VPR):
                mx = jnp.maximum(mx, au[j])
                sa = sa + au[j]
            maxu = _lane_max(mx)
            sumau = _lane_sum(sa)

            t = EPS * (maxu * math.sqrt(float(DIM)) + EPS)
            grms = maxu * (1.0 / _MAXC)
            d1 = grms + t
            _, num1, den1 = _quant_pass(u, bvec * d1, cvec, keep=False)
            g1p = num1 / (den1 + EPS)
            d2 = g1p + t
            spiky = maxu > 5.0 * (sumau * (1.0 / DIM) + t)
            df = jnp.where(spiky, d1, d2)
            rec, num2, den2 = _quant_pass(u, bvec * df, cvec, keep=True)
            gain = jnp.where(spiky, grms, num2 / (den2 + EPS))

            w = _fwht_regs(rec)
            for j in range(VPR):
                outb[pl.ds(ro + L * j, L)] = w[j] * sgn_s[j] * gain
            return carry

        def chunk_body(ci, carry):
            off = base + ci * (ch * DIM)
            pltpu.sync_copy(x_hbm.at[pl.ds(off, ch * DIM)], inb)
            lax.fori_loop(0, ch, row_body, 0)
            pltpu.sync_copy(outb, o_hbm.at[pl.ds(off, ch * DIM)])
            return carry

        lax.fori_loop(0, nch, chunk_body, 0)

    return sc_fn


def kernel(x):
    shape = x.shape
    rows = x.size // DIM
    xf = x.astype(jnp.float32).reshape(rows, DIM)
    rows_sc = (rows // 2) // (NW * 64) * (NW * 64)
    if rows_sc == 0:
        rows_sc = rows
    rpw = rows_sc // NW
    ch = 64 if rpw % 64 == 0 else rpw
    rows_tc = rows - rows_sc
    out_tc = None
    if rows_tc:
        tile = 1024 if rows_tc % 1024 == 0 else rows_tc
        out_tc = _make_tc_call(rows_tc, tile)(xf[rows_sc:], jnp.asarray(_M))
    out_sc = _make_sc_call(rows_sc, ch)(
        xf[:rows_sc].reshape(-1), jnp.asarray(_CONSTS))
    parts = [out_sc.reshape(rows_sc, DIM)]
    if out_tc is not None:
        parts.append(out_tc)
    out = jnp.concatenate(parts, axis=0) if len(parts) > 1 else parts[0]
    return out.reshape(shape)


# speedup vs baseline: 1.2684x; 1.0672x over previous
import functools
import math

import numpy as np
import jax
import jax.numpy as jnp
from jax import lax
from jax.experimental import pallas as pl
from jax.experimental.pallas import tpu as pltpu
from jax.experimental.pallas import tpu_sc as plsc

DIM = 128
BITS = 4
EPS = 1e-08
L = 16
VPR = DIM // L
NC, NS = 2, 16
NW = NC * NS


def _lm_centroids(bits, iters=100):
    n = 2 ** bits
    xs = np.linspace(-8.0, 8.0, 200001)
    pdf = np.exp(-0.5 * xs ** 2)
    cdf = np.cumsum(pdf)
    cdf = cdf / cdf[-1]
    c = np.interp((np.arange(n) + 0.5) / n, cdf, xs)
    for _ in range(iters):
        b = 0.5 * (c[:-1] + c[1:])
        idx = np.searchsorted(b, xs)
        num = np.bincount(idx, weights=pdf * xs, minlength=n)
        den = np.bincount(idx, weights=pdf, minlength=n)
        c = np.where(den > 1e-12, num / np.maximum(den, 1e-12), c)
    return c.astype(np.float32)


_CEN = _lm_centroids(BITS)
_BND = (0.5 * (_CEN[:-1] + _CEN[1:])).astype(np.float32)
_BND16 = np.concatenate([_BND, [np.float32(np.inf)]])
_MAXC = float(_CEN.max())
_SGN = (np.random.RandomState(42).randint(0, 2, (1, DIM)) * 2 - 1).astype(
    np.float32)[0]
_S = 1.0 / math.sqrt(float(DIM))

_CONSTS = np.concatenate(
    [_CEN, _BND16, (_SGN * np.float32(_S)).astype(np.float32)]
).astype(np.float32)

_GDN = lax.GatherDimensionNumbers(
    offset_dims=(), collapsed_slice_dims=(0,), start_index_map=(0,))


def _dg(vec, idx):
    return lax.gather(vec, idx[:, None], _GDN, (1,),
                      mode=lax.GatherScatterMode.PROMISE_IN_BOUNDS)


def _lane_sum(v):
    lane = lax.iota(jnp.int32, L)
    for h in (1, 2, 4, 8):
        v = v + _dg(v, lane ^ h)
    return v


def _lane_max(v):
    lane = lax.iota(jnp.int32, L)
    for h in (1, 2, 4, 8):
        v = jnp.maximum(v, _dg(v, lane ^ h))
    return v


def _fwht_regs(v):
    lane = lax.iota(jnp.int32, L)
    for h in (1, 2, 4, 8):
        perm = lane ^ h
        pm = jnp.where((lane & h) == 0, 1.0, -1.0).astype(jnp.float32)
        v = [_dg(x, perm) + pm * x for x in v]
    for hb in (1, 2, 4):
        nv = list(v)
        for j in range(VPR):
            if j & hb == 0:
                nv[j] = v[j] + v[j ^ hb]
            else:
                nv[j] = v[j ^ hb] - v[j]
        v = nv
    return v


def _quant_pass(u, bs, cvec, keep):
    recon = [] if keep else None
    num_acc = None
    den_acc = None
    for j in range(VPR):
        idx = jnp.zeros((L,), jnp.int32)
        for stp in (8, 4, 2, 1):
            bv = _dg(bs, idx + (stp - 1))
            idx = idx + jnp.where(bv < u[j], stp, 0).astype(jnp.int32)
        r = _dg(cvec, idx)
        if keep:
            recon.append(r)
        nj = u[j] * r
        dj = r * r
        num_acc = nj if num_acc is None else num_acc + nj
        den_acc = dj if den_acc is None else den_acc + dj
    return recon, _lane_sum(num_acc), _lane_sum(den_acc)


_H = np.ones((1, 1), np.float32)
for _ in range(7):
    _H = np.block([[_H, _H], [_H, -_H]])
_M = (np.diag(_SGN) @ _H * np.float32(_S)).astype(np.float32)
_SQD = float(math.sqrt(float(DIM)))


def _tc_recon(un, d):
    r = jnp.full(un.shape, float(_CEN[0]), jnp.float32)
    for k in range(15):
        r = r + jnp.where(un > float(_BND[k]) * d,
                          float(_CEN[k + 1] - _CEN[k]), 0.0)
    return r


def _tc_body(x_ref, m_ref, o_ref):
    xt = x_ref[...]
    m = m_ref[...]
    u = jnp.dot(xt, m, preferred_element_type=jnp.float32,
                precision=lax.Precision.HIGHEST)
    au = jnp.abs(u)
    maxu = jnp.max(au, axis=-1, keepdims=True)
    sumau = jnp.sum(au, axis=-1, keepdims=True)
    t = EPS * (maxu * _SQD + EPS)
    grms = maxu * (1.0 / _MAXC)
    d1 = grms + t
    r1 = _tc_recon(u, d1)
    num1 = jnp.sum(u * r1, axis=-1, keepdims=True)
    den1 = jnp.sum(r1 * r1, axis=-1, keepdims=True)
    d2 = num1 / (den1 + EPS) + t
    spiky = maxu > 5.0 * (sumau * (1.0 / DIM) + t)
    df = jnp.where(spiky, d1, d2)
    rf = _tc_recon(u, df)
    num2 = jnp.sum(u * rf, axis=-1, keepdims=True)
    den2 = jnp.sum(rf * rf, axis=-1, keepdims=True)
    gain = jnp.where(spiky, grms, num2 / (den2 + EPS))
    w = jnp.dot(rf, m.T, preferred_element_type=jnp.float32,
                precision=lax.Precision.HIGHEST)
    o_ref[...] = w * gain


def _make_tc_call(rows, tile):
    grid = rows // tile
    return pl.pallas_call(
        _tc_body,
        grid=(grid,),
        in_specs=[
            pl.BlockSpec((tile, DIM), lambda i: (i, 0)),
            pl.BlockSpec((DIM, DIM), lambda i: (0, 0)),
        ],
        out_specs=pl.BlockSpec((tile, DIM), lambda i: (i, 0)),
        out_shape=jax.ShapeDtypeStruct((rows, DIM), jnp.float32),
    )


def _make_sc_call(rows, ch, interpret=False):
    rpw = rows // NW
    nch = rpw // ch
    mesh = plsc.VectorSubcoreMesh(core_axis_name="c", subcore_axis_name="s",
                                  num_cores=NC, num_subcores=NS)

    @functools.partial(
        pl.kernel,
        out_type=jax.ShapeDtypeStruct((rows * DIM,), jnp.float32),
        mesh=mesh,
        scratch_types=[
            pltpu.VMEM((_CONSTS.size,), jnp.float32),
            pltpu.VMEM((ch * DIM,), jnp.float32),
            pltpu.VMEM((ch * DIM,), jnp.float32),
        ],
        interpret=interpret,
    )
    def sc_fn(x_hbm, c_hbm, o_hbm, cbuf, inb, outb):
        wid = lax.axis_index("s") * NC + lax.axis_index("c")
        base = wid * rpw * DIM

        pltpu.sync_copy(c_hbm, cbuf)
        cvec = cbuf[pl.ds(0, L)]
        bvec = cbuf[pl.ds(L, L)]
        sgn_s = [cbuf[pl.ds(2 * L + L * j, L)] for j in range(VPR)]

        def row_body(i, carry):
            ro = i * DIM
            v = [inb[pl.ds(ro + L * j, L)] for j in range(VPR)]
            u = _fwht_regs([v[j] * sgn_s[j] for j in range(VPR)])
            au = [jnp.abs(t) for t in u]
            mx = au[0]
            sa = au[0]
            for j in range(1, VPR):
                mx = jnp.maximum(mx, au[j])
                sa = sa + au[j]
            maxu = _lane_max(mx)
            sumau = _lane_sum(sa)

            t = EPS * (maxu * math.sqrt(float(DIM)) + EPS)
            grms = maxu * (1.0 / _MAXC)
            d1 = grms + t
            _, num1, den1 = _quant_pass(u, bvec * d1, cvec, keep=False)
            g1p = num1 / (den1 + EPS)
            d2 = g1p + t
            spiky = maxu > 5.0 * (sumau * (1.0 / DIM) + t)
            df = jnp.where(spiky, d1, d2)
            rec, num2, den2 = _quant_pass(u, bvec * df, cvec, keep=True)
            gain = jnp.where(spiky, grms, num2 / (den2 + EPS))

            w = _fwht_regs(rec)
            for j in range(VPR):
                outb[pl.ds(ro + L * j, L)] = w[j] * sgn_s[j] * gain
            return carry

        def chunk_body(ci, carry):
            off = base + ci * (ch * DIM)
            pltpu.sync_copy(x_hbm.at[pl.ds(off, ch * DIM)], inb)
            lax.fori_loop(0, ch, row_body, 0)
            pltpu.sync_copy(outb, o_hbm.at[pl.ds(off, ch * DIM)])
            return carry

        lax.fori_loop(0, nch, chunk_body, 0)

    return sc_fn


def kernel(x):
    shape = x.shape
    rows = x.size // DIM
    xf = x.astype(jnp.float32).reshape(rows, DIM)
    rows_sc = (rows * 7 // 16) // (NW * 64) * (NW * 64)
    if rows_sc == 0:
        rows_sc = rows
    rpw = rows_sc // NW
    ch = 64 if rpw % 64 == 0 else rpw
    rows_tc = rows - rows_sc
    out_tc = None
    if rows_tc:
        tile = 1024 if rows_tc % 1024 == 0 else rows_tc
        out_tc = _make_tc_call(rows_tc, tile)(xf[rows_sc:], jnp.asarray(_M))
    out_sc = _make_sc_call(rows_sc, ch)(
        xf[:rows_sc].reshape(-1), jnp.asarray(_CONSTS))
    parts = [out_sc.reshape(rows_sc, DIM)]
    if out_tc is not None:
        parts.append(out_tc)
    out = jnp.concatenate(parts, axis=0) if len(parts) > 1 else parts[0]
    return out.reshape(shape)
